# Initial kernel scaffold; baseline (speedup 1.0000x reference)
import jax
import jax.numpy as jnp
from jax.experimental import pallas as pl


def kernel(static_cont_input, static_cat_input, history_cont_input, history_cat_input, future_input, W_s, b_s, W_h, b_h, static_tables, history_tables, future_tables):
    raise NotImplementedError("write your pallas kernel here")


# speedup vs baseline: 1.0166x; 1.0166x over previous
import functools

import jax
import jax.numpy as jnp
from jax import lax
from jax.experimental import pallas as pl
from jax.experimental.pallas import tpu as pltpu
from jax.experimental.pallas import tpu_sc as plsc

NC = 2
NS = 16
NW = NC * NS
H = 128


def _matmul_body(x_ref, w_ref, b_ref, o_ref):
    o_ref[...] = (
        jnp.dot(x_ref[...], w_ref[...], preferred_element_type=jnp.float32)
        + b_ref[...]
    )


def _linear(x, w, b, bm):
    m = x.shape[0]
    return pl.pallas_call(
        _matmul_body,
        grid=(m // bm,),
        in_specs=[
            pl.BlockSpec((bm, 16), lambda i: (i, 0)),
            pl.BlockSpec((16, H), lambda i: (0, 0)),
            pl.BlockSpec((1, H), lambda i: (0, 0)),
        ],
        out_specs=pl.BlockSpec((bm, H), lambda i: (i, 0)),
        out_shape=jax.ShapeDtypeStruct((m, H), jnp.float32),
    )(x, w, b.reshape(1, H))


def _make_sc_interleave(n_pairs, c_cat, has_cont, ch):
    cw = c_cat + (1 if has_cont else 0)
    ppt = n_pairs // NW
    n_chunks = ppt // ch
    irpc = (ch * c_cat) // 128
    assert ppt * NW == n_pairs and n_chunks * ch == ppt and irpc * 128 == ch * c_cat

    mesh = plsc.VectorSubcoreMesh(core_axis_name="c", subcore_axis_name="s")
    scratch = [
        pltpu.VMEM((ppt * c_cat,), jnp.int32),
        pltpu.VMEM((ch * c_cat, H), jnp.float32),
        pltpu.VMEM((ch, cw * H), jnp.float32),
        pltpu.SemaphoreType.DMA,
    ]
    if has_cont:
        scratch.insert(2, pltpu.VMEM((ch, H), jnp.float32))

    out_type = jax.ShapeDtypeStruct((n_pairs, cw * H), jnp.float32)

    @functools.partial(
        pl.kernel, mesh=mesh, out_type=out_type, scratch_types=scratch,
        compiler_params=pltpu.CompilerParams(needs_layout_passes=False))
    def sc_kernel(*refs):
        if has_cont:
            tab, idx, cont, out, idx_v, rows_v, cont_v, outbuf, sem = refs
        else:
            tab, idx, out, idx_v, rows_v, outbuf, sem = refs
            cont_v = None
        wid = lax.axis_index("s") * NC + lax.axis_index("c")
        pair0 = wid * ppt
        iota = lax.iota(jnp.int32, 16)
        lanes = [iota + 16 * i for i in range(H // 16)]
        lanes_cw = [l * cw for l in lanes]
        pltpu.sync_copy(idx.at[pl.ds(pair0 * c_cat, ppt * c_cat)], idx_v)

        def chunk_body(g, carry):
            base = pair0 + g * ch
            handles = [
                pltpu.async_copy(
                    tab.at[idx_v.at[pl.ds(g * ch * c_cat + j * 128, 128)]],
                    rows_v.at[pl.ds(j * 128, 128)], sem)
                for j in range(irpc)
            ]
            if has_cont:
                pltpu.sync_copy(cont.at[pl.ds(base, ch)], cont_v)
            for hnd in handles:
                hnd.wait()

            def pair_body(p, c2):
                prow = jnp.full((16,), p, jnp.int32)
                rowbase = p * c_cat
                for i in range(H // 16):
                    if has_cont:
                        v = plsc.load_gather(cont_v, [prow, lanes[i]])
                        plsc.store_scatter(outbuf, [prow, lanes_cw[i]], v)
                    for c in range(c_cat):
                        rsplat = jnp.full((16,), rowbase + c, jnp.int32)
                        v = plsc.load_gather(rows_v, [rsplat, lanes[i]])
                        off = c + 1 if has_cont else c
                        plsc.store_scatter(outbuf, [prow, lanes_cw[i] + off], v)
                return c2

            lax.fori_loop(0, ch, pair_body, 0)
            pltpu.sync_copy(outbuf, out.at[pl.ds(base, ch)])
            return carry

        lax.fori_loop(0, n_chunks, chunk_body, 0)

    return sc_kernel


def kernel(static_cont_input, static_cat_input, history_cont_input,
           history_cat_input, future_input, W_s, b_s, W_h, b_h,
           static_tables, history_tables, future_tables):
    B, T_h, _ = history_cont_input.shape
    T_f = future_input.shape[1]

    static_cont_emb = _linear(static_cont_input, W_s, b_s, bm=B)
    hist_cont_emb = _linear(history_cont_input.reshape(B * T_h, 16),
                            W_h, b_h, bm=1024)

    def prep_idx(cat, n_pairs, c_cat, v_rows):
        offs = jnp.arange(c_cat, dtype=jnp.int32) * v_rows
        flat = cat.reshape(n_pairs, c_cat).astype(jnp.int32) + offs
        return flat.reshape(n_pairs * c_cat)

    idx_s = prep_idx(static_cat_input, B, 4, 10000)
    idx_h = prep_idx(history_cat_input, B * T_h, 4, 1000)
    idx_f = prep_idx(future_input, B * T_f, 3, 1000)

    tab_s = static_tables.reshape(4 * 10000, H)
    tab_h = history_tables.reshape(4 * 1000, H)
    tab_f = future_tables.reshape(3 * 1000, H)

    static_out = _make_sc_interleave(B, 4, True, ch=32)(
        tab_s, idx_s, static_cont_emb)
    hist_out = _make_sc_interleave(B * T_h, 4, True, ch=64)(
        tab_h, idx_h, hist_cont_emb)
    fut_out = _make_sc_interleave(B * T_f, 3, False, ch=128)(
        tab_f, idx_f)

    return (static_out.reshape(B, H, 5),
            hist_out.reshape(B, T_h, H, 5),
            fut_out.reshape(B, T_f, H, 3))


# speedup vs baseline: 4.8100x; 4.7316x over previous
import functools

import jax
import jax.numpy as jnp
from jax import lax
from jax.experimental import pallas as pl
from jax.experimental.pallas import tpu as pltpu
from jax.experimental.pallas import tpu_sc as plsc

NC = 2
NS = 16
NW = NC * NS
H = 128


def _matmul_body(x_ref, w_ref, b_ref, o_ref):
    o_ref[...] = (
        jnp.dot(x_ref[...], w_ref[...], preferred_element_type=jnp.float32)
        + b_ref[...]
    )


def _linear(x, w, b, bm):
    m = x.shape[0]
    return pl.pallas_call(
        _matmul_body,
        grid=(m // bm,),
        in_specs=[
            pl.BlockSpec((bm, 16), lambda i: (i, 0)),
            pl.BlockSpec((16, H), lambda i: (0, 0)),
            pl.BlockSpec((1, H), lambda i: (0, 0)),
        ],
        out_specs=pl.BlockSpec((bm, H), lambda i: (i, 0)),
        out_shape=jax.ShapeDtypeStruct((m, H), jnp.float32),
    )(x, w, b.reshape(1, H))


def _matmul_concat_body(x_ref, w_ref, b_ref, t_ref, o_ref, *, nmm):
    pid = pl.program_id(0)

    @pl.when(pid < nmm)
    def _():
        o_ref[...] = (
            jnp.dot(x_ref[...], w_ref[...],
                    preferred_element_type=jnp.float32)
            + b_ref[...]
        )

    @pl.when(pid >= nmm)
    def _():
        o_ref[...] = t_ref[...]


def _linear_concat(x, w, b, tab, bm):
    m = x.shape[0]
    mt = tab.shape[0]
    nmm = m // bm
    nt = mt // bm
    assert nmm * bm == m and nt * bm == mt
    return pl.pallas_call(
        functools.partial(_matmul_concat_body, nmm=nmm),
        grid=(nmm + nt,),
        in_specs=[
            pl.BlockSpec((bm, 16), lambda i: (jnp.minimum(i, nmm - 1), 0)),
            pl.BlockSpec((16, H), lambda i: (0, 0)),
            pl.BlockSpec((1, H), lambda i: (0, 0)),
            pl.BlockSpec((bm, H), lambda i: (jnp.maximum(i - nmm, 0), 0)),
        ],
        out_specs=pl.BlockSpec((bm, H), lambda i: (i, 0)),
        out_shape=jax.ShapeDtypeStruct((m + mt, H), jnp.float32),
    )(x, w, b.reshape(1, H), tab)


def _make_sc_gather(n_rows):
    rpt = n_rows // NW
    assert rpt * NW == n_rows and rpt % 8 == 0
    n_full, rem = divmod(rpt, 128)
    assert rem % 8 == 0

    mesh = plsc.VectorSubcoreMesh(core_axis_name="c", subcore_axis_name="s")
    scratch = [pltpu.VMEM((rpt,), jnp.int32)]
    for _ in range(2):
        scratch += [pltpu.VMEM((128, H), jnp.float32),
                    pltpu.SemaphoreType.DMA, pltpu.SemaphoreType.DMA]

    @functools.partial(
        pl.kernel, mesh=mesh,
        out_type=jax.ShapeDtypeStruct((n_rows, H), jnp.float32),
        scratch_types=scratch,
        compiler_params=pltpu.CompilerParams(needs_layout_passes=False))
    def sc_kernel(src, idx, out, idx_v, r0, g0, o0, r1, g1, o1):
        wid = lax.axis_index("s") * NC + lax.axis_index("c")
        base = wid * rpt
        bufs = ((r0, g0, o0), (r1, g1, o1))
        pltpu.sync_copy(idx.at[pl.ds(base, rpt)], idx_v)

        def g_copy(off, sz, bd, make_only):
            mk = pltpu.make_async_copy if make_only else pltpu.async_copy
            return mk(src.at[idx_v.at[pl.ds(off, sz)]],
                      bd[0].at[pl.ds(0, sz)], bd[1])

        def o_copy(off, sz, bd, make_only):
            mk = pltpu.make_async_copy if make_only else pltpu.async_copy
            return mk(bd[0].at[pl.ds(0, sz)],
                      out.at[pl.ds(base + off, sz)], bd[2])

        def process(k, sz, bd):
            off = k * 128
            g_copy(off, sz, bd, True).wait()
            o_copy(off, sz, bd, False)
            o_copy(off, sz, bd, True).wait()

        if n_full >= 1:
            g_copy(0, 128, bufs[0], False)
        if n_full >= 2:
            g_copy(128, 128, bufs[1], False)

        def body(i, carry):
            for par in range(2):
                k = 2 * i + par
                bd = bufs[par]
                process(k, 128, bd)

                @pl.when(k + 2 < n_full)
                def _():
                    g_copy((k + 2) * 128, 128, bd, False)
            return carry

        lax.fori_loop(0, n_full // 2, body, 0)

        if n_full % 2 == 1:
            process(n_full - 1, 128, bufs[(n_full - 1) % 2])
        if rem:
            bd = bufs[n_full % 2]
            g_copy(n_full * 128, rem, bd, False)
            process(n_full, rem, bd)

    return sc_kernel


def kernel(static_cont_input, static_cat_input, history_cont_input,
           history_cat_input, future_input, W_s, b_s, W_h, b_h,
           static_tables, history_tables, future_tables):
    B, T_h, _ = history_cont_input.shape
    T_f = future_input.shape[1]
    i32 = jnp.int32

    tab_s = static_tables.reshape(4 * 10000, H)
    tab_h = history_tables.reshape(4 * 1000, H)
    tab_f = future_tables.reshape(3 * 1000, H)

    static_cont_emb = _linear(static_cont_input, W_s, b_s, bm=B)
    src_h = _linear_concat(history_cont_input.reshape(B * T_h, 16),
                           W_h, b_h, tab_h, bm=400)

    idx0_h = (jnp.arange(B, dtype=i32) * T_h)[None, :] \
        + jnp.arange(T_h, dtype=i32)[:, None]
    cat_h = history_cat_input.astype(i32).transpose(1, 2, 0)
    off_h = (jnp.arange(4, dtype=i32) * 1000 + B * T_h).reshape(1, 4, 1)
    idx_h = jnp.concatenate([idx0_h[:, None, :], cat_h + off_h],
                            axis=1).reshape(-1)
    idx_f = (future_input.astype(i32).transpose(1, 2, 0)
             + (jnp.arange(3, dtype=i32) * 1000).reshape(1, 3, 1)
             ).reshape(-1)
    idx_s = (static_cat_input.astype(i32).T
             + (jnp.arange(4, dtype=i32) * 10000).reshape(4, 1)
             ).reshape(-1)

    hist_rows = _make_sc_gather(T_h * 5 * B)(src_h, idx_h)
    fut_rows = _make_sc_gather(T_f * 3 * B)(tab_f, idx_f)
    stat_rows = _make_sc_gather(4 * B)(tab_s, idx_s)

    static_out = jnp.concatenate(
        [static_cont_emb[None], stat_rows.reshape(4, B, H)], axis=0
    ).transpose(1, 2, 0)
    hist_out = hist_rows.reshape(T_h, 5, B, H).transpose(2, 0, 3, 1)
    fut_out = fut_rows.reshape(T_f, 3, B, H).transpose(2, 0, 3, 1)

    return (static_out, hist_out, fut_out)


# speedup vs baseline: 5.0774x; 1.0556x over previous
import functools

import jax
import jax.numpy as jnp
from jax import lax
from jax.experimental import pallas as pl
from jax.experimental.pallas import tpu as pltpu
from jax.experimental.pallas import tpu_sc as plsc

NC = 2
NS = 16
NW = NC * NS
H = 128


def _matmul_body(x_ref, w_ref, b_ref, o_ref):
    o_ref[...] = (
        jnp.dot(x_ref[...], w_ref[...], preferred_element_type=jnp.float32)
        + b_ref[...]
    )


def _linear(x, w, b, bm):
    m = x.shape[0]
    return pl.pallas_call(
        _matmul_body,
        grid=(m // bm,),
        in_specs=[
            pl.BlockSpec((bm, 16), lambda i: (i, 0)),
            pl.BlockSpec((16, H), lambda i: (0, 0)),
            pl.BlockSpec((1, H), lambda i: (0, 0)),
        ],
        out_specs=pl.BlockSpec((bm, H), lambda i: (i, 0)),
        out_shape=jax.ShapeDtypeStruct((m, H), jnp.float32),
    )(x, w, b.reshape(1, H))


def _matmul_concat_body(x_ref, w_ref, b_ref, t_ref, o_ref, *, nmm):
    pid = pl.program_id(0)

    @pl.when(pid < nmm)
    def _():
        o_ref[...] = (
            jnp.dot(x_ref[...], w_ref[...],
                    preferred_element_type=jnp.float32)
            + b_ref[...]
        )

    @pl.when(pid >= nmm)
    def _():
        o_ref[...] = t_ref[...]


def _linear_concat(x, w, b, tab, bm):
    m = x.shape[0]
    mt = tab.shape[0]
    nmm = m // bm
    nt = mt // bm
    assert nmm * bm == m and nt * bm == mt
    return pl.pallas_call(
        functools.partial(_matmul_concat_body, nmm=nmm),
        grid=(nmm + nt,),
        in_specs=[
            pl.BlockSpec((bm, 16), lambda i: (jnp.minimum(i, nmm - 1), 0)),
            pl.BlockSpec((16, H), lambda i: (0, 0)),
            pl.BlockSpec((1, H), lambda i: (0, 0)),
            pl.BlockSpec((bm, H), lambda i: (jnp.maximum(i - nmm, 0), 0)),
        ],
        out_specs=pl.BlockSpec((bm, H), lambda i: (i, 0)),
        out_shape=jax.ShapeDtypeStruct((m + mt, H), jnp.float32),
    )(x, w, b.reshape(1, H), tab)


def _make_sc_gather_all(n_h, n_f, n_s):
    rpt_h, rpt_f, rpt_s = n_h // NW, n_f // NW, n_s // NW
    mesh = plsc.VectorSubcoreMesh(core_axis_name="c", subcore_axis_name="s")
    scratch = [
        pltpu.VMEM_SHARED((3000, H), jnp.float32),
        pltpu.VMEM((rpt_h,), jnp.int32),
        pltpu.VMEM((rpt_f,), jnp.int32),
        pltpu.VMEM((rpt_s,), jnp.int32),
    ]
    for _ in range(2):
        scratch += [pltpu.VMEM((128, H), jnp.float32),
                    pltpu.SemaphoreType.DMA, pltpu.SemaphoreType.DMA]

    out_type = [jax.ShapeDtypeStruct((n_h, H), jnp.float32),
                jax.ShapeDtypeStruct((n_f, H), jnp.float32),
                jax.ShapeDtypeStruct((n_s, H), jnp.float32)]

    @functools.partial(
        pl.kernel, mesh=mesh, out_type=out_type, scratch_types=scratch,
        compiler_params=pltpu.CompilerParams(needs_layout_passes=False))
    def sc_kernel(src_h, idx_h, tab_f, idx_f, tab_s, idx_s,
                  out_h, out_f, out_s, shr_f, ixh, ixf, ixs,
                  r0, g0, o0, r1, g1, o1):
        sid = lax.axis_index("s")
        wid = sid * NC + lax.axis_index("c")
        bufs = ((r0, g0, o0), (r1, g1, o1))

        @pl.when(sid == 0)
        def _():
            pltpu.sync_copy(tab_f, shr_f)

        pltpu.sync_copy(idx_h.at[pl.ds(wid * rpt_h, rpt_h)], ixh)
        pltpu.sync_copy(idx_f.at[pl.ds(wid * rpt_f, rpt_f)], ixf)
        pltpu.sync_copy(idx_s.at[pl.ds(wid * rpt_s, rpt_s)], ixs)
        plsc.subcore_barrier()

        def run_phase(rpt, ixv, out, src):
            base = wid * rpt
            n_full, rem = divmod(rpt, 128)

            def g_copy(off, sz, bd, make_only):
                mk = pltpu.make_async_copy if make_only else pltpu.async_copy
                return mk(src.at[ixv.at[pl.ds(off, sz)]],
                          bd[0].at[pl.ds(0, sz)], bd[1])

            def o_copy(off, sz, bd, make_only):
                mk = pltpu.make_async_copy if make_only else pltpu.async_copy
                return mk(bd[0].at[pl.ds(0, sz)],
                          out.at[pl.ds(base + off, sz)], bd[2])

            def process(k, sz, bd):
                off = k * 128
                g_copy(off, sz, bd, True).wait()
                o_copy(off, sz, bd, False)
                o_copy(off, sz, bd, True).wait()

            if n_full >= 1:
                g_copy(0, 128, bufs[0], False)
            if n_full >= 2:
                g_copy(128, 128, bufs[1], False)

            def body(i, carry):
                for par in range(2):
                    k = 2 * i + par
                    bd = bufs[par]
                    process(k, 128, bd)

                    @pl.when(k + 2 < n_full)
                    def _():
                        g_copy((k + 2) * 128, 128, bd, False)
                return carry

            lax.fori_loop(0, n_full // 2, body, 0)

            if n_full % 2 == 1:
                process(n_full - 1, 128, bufs[(n_full - 1) % 2])
            if rem:
                bd = bufs[n_full % 2]
                g_copy(n_full * 128, rem, bd, False)
                process(n_full, rem, bd)

        run_phase(rpt_h, ixh, out_h, src_h)
        run_phase(rpt_f, ixf, out_f, shr_f)
        run_phase(rpt_s, ixs, out_s, tab_s)

    return sc_kernel


def kernel(static_cont_input, static_cat_input, history_cont_input,
           history_cat_input, future_input, W_s, b_s, W_h, b_h,
           static_tables, history_tables, future_tables):
    B, T_h, _ = history_cont_input.shape
    T_f = future_input.shape[1]
    i32 = jnp.int32

    tab_s = static_tables.reshape(4 * 10000, H)
    tab_h = history_tables.reshape(4 * 1000, H)
    tab_f = future_tables.reshape(3 * 1000, H)

    static_cont_emb = _linear(static_cont_input, W_s, b_s, bm=B)
    src_h = _linear_concat(history_cont_input.reshape(B * T_h, 16),
                           W_h, b_h, tab_h, bm=400)

    idx0_h = (jnp.arange(B, dtype=i32) * T_h)[None, :] \
        + jnp.arange(T_h, dtype=i32)[:, None]
    cat_h = history_cat_input.astype(i32).transpose(1, 2, 0)
    off_h = (jnp.arange(4, dtype=i32) * 1000 + B * T_h).reshape(1, 4, 1)
    idx_h = jnp.concatenate([idx0_h[:, None, :], cat_h + off_h],
                            axis=1).reshape(-1)
    idx_f = (future_input.astype(i32).transpose(1, 2, 0)
             + (jnp.arange(3, dtype=i32) * 1000).reshape(1, 3, 1)
             ).reshape(-1)
    idx_s = (static_cat_input.astype(i32).T
             + (jnp.arange(4, dtype=i32) * 10000).reshape(4, 1)
             ).reshape(-1)

    hist_rows, fut_rows, stat_rows = _make_sc_gather_all(
        T_h * 5 * B, T_f * 3 * B, 4 * B)(
        src_h, idx_h, tab_f, idx_f, tab_s, idx_s)

    static_out = jnp.concatenate(
        [static_cont_emb[None], stat_rows.reshape(4, B, H)], axis=0
    ).transpose(1, 2, 0)
    hist_out = hist_rows.reshape(T_h, 5, B, H).transpose(2, 0, 3, 1)
    fut_out = fut_rows.reshape(T_f, 3, B, H).transpose(2, 0, 3, 1)

    return (static_out, hist_out, fut_out)


# speedup vs baseline: 7.4544x; 1.4681x over previous
import functools

import jax
import jax.numpy as jnp
from jax import lax
from jax.experimental import pallas as pl
from jax.experimental.pallas import tpu as pltpu
from jax.experimental.pallas import tpu_sc as plsc

NC = 2
NS = 16
NW = NC * NS
H = 128


def _matmul_body(x_ref, w_ref, b_ref, o_ref):
    o_ref[...] = (
        jnp.dot(x_ref[...], w_ref[...], preferred_element_type=jnp.float32)
        + b_ref[...]
    )


def _linear(x, w, b, bm):
    m = x.shape[0]
    return pl.pallas_call(
        _matmul_body,
        grid=(m // bm,),
        in_specs=[
            pl.BlockSpec((bm, 16), lambda i: (i, 0)),
            pl.BlockSpec((16, H), lambda i: (0, 0)),
            pl.BlockSpec((1, H), lambda i: (0, 0)),
        ],
        out_specs=pl.BlockSpec((bm, H), lambda i: (i, 0)),
        out_shape=jax.ShapeDtypeStruct((m, H), jnp.float32),
    )(x, w, b.reshape(1, H))


def _matmul_t_concat_body(x_ref, w_ref, b_ref, t_ref, o_ref, *, nmm):
    pid = pl.program_id(0)

    @pl.when(pid < nmm)
    def _():
        o_ref[0] = lax.dot_general(
            x_ref[0], w_ref[...], (((0,), (0,)), ((), ())),
            preferred_element_type=jnp.float32) + b_ref[...]

    @pl.when(pid >= nmm)
    def _():
        o_ref[...] = t_ref[...]


def _linear_t_concat(x_t, w, b, tab3):
    t_dim, _, b_dim = x_t.shape
    p_dim = tab3.shape[0]
    return pl.pallas_call(
        functools.partial(_matmul_t_concat_body, nmm=t_dim),
        grid=(t_dim + p_dim,),
        in_specs=[
            pl.BlockSpec((1, 16, b_dim),
                         lambda i: (jnp.minimum(i, t_dim - 1), 0, 0)),
            pl.BlockSpec((16, H), lambda i: (0, 0)),
            pl.BlockSpec((1, H), lambda i: (0, 0)),
            pl.BlockSpec((1, b_dim, H),
                         lambda i: (jnp.maximum(i - t_dim, 0), 0, 0)),
        ],
        out_specs=pl.BlockSpec((1, b_dim, H), lambda i: (i, 0, 0)),
        out_shape=jax.ShapeDtypeStruct((t_dim + p_dim, b_dim, H),
                                       jnp.float32),
    )(x_t, w, b.reshape(1, H), tab3)


def _phase_runner(wid, bufs):
    def run_phase(rpt, ixv, out, fire, wait_do):
            base = wid * rpt
            n_full, rem = divmod(rpt, 128)

            def o_copy(off, sz, bd, make_only):
                mk = pltpu.make_async_copy if make_only else pltpu.async_copy
                return mk(bd[0].at[pl.ds(0, sz)],
                          out.at[pl.ds(base + off, sz)], bd[2])

            def process(k, sz, bd):
                off = k * 128
                wait_do(off, sz, bd)
                o_copy(off, sz, bd, False)
                o_copy(off, sz, bd, True).wait()

            if n_full >= 1:
                fire(0, 128, bufs[0])
            if n_full >= 2:
                fire(128, 128, bufs[1])

            def body(i, carry):
                for par in range(2):
                    k = 2 * i + par
                    bd = bufs[par]
                    process(k, 128, bd)

                    @pl.when(k + 2 < n_full)
                    def _():
                        fire((k + 2) * 128, 128, bd)
                return carry

            lax.fori_loop(0, n_full // 2, body, 0)

            if n_full % 2 == 1:
                process(n_full - 1, 128, bufs[(n_full - 1) % 2])
            if rem:
                bd = bufs[n_full % 2]
                fire(n_full * 128, rem, bd)
                process(n_full, rem, bd)

    return run_phase


def _simple_src(src, ixv):
    def fire(off, sz, bd):
        pltpu.async_copy(src.at[ixv.at[pl.ds(off, sz)]],
                         bd[0].at[pl.ds(0, sz)], bd[1])

    def wait_do(off, sz, bd):
        pltpu.make_async_copy(src.at[ixv.at[pl.ds(off, sz)]],
                              bd[0].at[pl.ds(0, sz)], bd[1]).wait()
    return fire, wait_do


def _sc_scratch(idx_sizes, shared_rows):
    scratch = [pltpu.VMEM_SHARED((r, H), jnp.float32) for r in shared_rows]
    scratch += [pltpu.VMEM((n,), jnp.int32) for n in idx_sizes]
    for _ in range(2):
        scratch += [pltpu.VMEM((128, H), jnp.float32),
                    pltpu.SemaphoreType.DMA, pltpu.SemaphoreType.DMA]
    return scratch


def _make_sc_gather_fs(n_f, n_s):
    rpt_f, rpt_s = n_f // NW, n_s // NW
    mesh = plsc.VectorSubcoreMesh(core_axis_name="c", subcore_axis_name="s")
    out_type = [jax.ShapeDtypeStruct((n_f, H), jnp.float32),
                jax.ShapeDtypeStruct((n_s, H), jnp.float32)]

    @functools.partial(
        pl.kernel, mesh=mesh, out_type=out_type,
        scratch_types=_sc_scratch([rpt_f, rpt_s], [3000]),
        compiler_params=pltpu.CompilerParams(needs_layout_passes=False))
    def sc_kernel(tab_f, idx_f, tab_s, idx_s, out_f, out_s,
                  shr_f, ixf, ixs, r0, g0, o0, r1, g1, o1):
        sid = lax.axis_index("s")
        wid = sid * NC + lax.axis_index("c")
        bufs = ((r0, g0, o0), (r1, g1, o1))

        @pl.when(sid == 0)
        def _():
            pltpu.sync_copy(tab_f, shr_f)

        pltpu.sync_copy(idx_f.at[pl.ds(wid * rpt_f, rpt_f)], ixf)
        pltpu.sync_copy(idx_s.at[pl.ds(wid * rpt_s, rpt_s)], ixs)
        plsc.subcore_barrier()

        run_phase = _phase_runner(wid, bufs)
        run_phase(rpt_f, ixf, out_f, *_simple_src(shr_f, ixf))
        run_phase(rpt_s, ixs, out_s, *_simple_src(tab_s, ixs))

    return sc_kernel


def _make_sc_gather_h(n_h, n_src):
    rpt_h = n_h // NW
    mesh = plsc.VectorSubcoreMesh(core_axis_name="c", subcore_axis_name="s")

    @functools.partial(
        pl.kernel, mesh=mesh,
        out_type=jax.ShapeDtypeStruct((n_h, H), jnp.float32),
        scratch_types=_sc_scratch([rpt_h], []),
        compiler_params=pltpu.CompilerParams(needs_layout_passes=False))
    def sc_kernel(src_h, idx_h, out_h, ixh, r0, g0, o0, r1, g1, o1):
        sid = lax.axis_index("s")
        wid = sid * NC + lax.axis_index("c")
        bufs = ((r0, g0, o0), (r1, g1, o1))
        pltpu.sync_copy(idx_h.at[pl.ds(wid * rpt_h, rpt_h)], ixh)
        run_phase = _phase_runner(wid, bufs)
        run_phase(rpt_h, ixh, out_h, *_simple_src(src_h, ixh))

    return sc_kernel


def kernel(static_cont_input, static_cat_input, history_cont_input,
           history_cat_input, future_input, W_s, b_s, W_h, b_h,
           static_tables, history_tables, future_tables):
    B, T_h, _ = history_cont_input.shape
    T_f = future_input.shape[1]
    i32 = jnp.int32

    tab_s = static_tables.reshape(4 * 10000, H)
    tab_h = history_tables.reshape(4 * 1000, H)
    tab_f = future_tables.reshape(3 * 1000, H)

    idx_f = (future_input.astype(i32).transpose(1, 2, 0)
             + (jnp.arange(3, dtype=i32) * 1000).reshape(1, 3, 1)
             ).reshape(-1)
    idx_s = (static_cat_input.astype(i32).T
             + (jnp.arange(4, dtype=i32) * 10000).reshape(4, 1)
             ).reshape(-1)

    fut_rows, stat_rows = _make_sc_gather_fs(T_f * 3 * B, 4 * B)(
        tab_f, idx_f, tab_s, idx_s)

    static_cont_emb = _linear(static_cont_input, W_s, b_s, bm=B)
    tab_h3 = jnp.pad(tab_h, ((0, 5 * B - 4000), (0, 0))).reshape(5, B, H)
    src_h = _linear_t_concat(history_cont_input.transpose(1, 2, 0),
                             W_h, b_h, tab_h3).reshape((T_h + 5) * B, H)

    idx0_h = (jnp.arange(T_h, dtype=i32) * B)[:, None] \
        + jnp.arange(B, dtype=i32)[None, :]
    cat_h = history_cat_input.astype(i32).transpose(1, 2, 0)
    off_h = (jnp.arange(4, dtype=i32) * 1000 + T_h * B).reshape(1, 4, 1)
    idx_h = jnp.concatenate([idx0_h[:, None, :], cat_h + off_h],
                            axis=1).reshape(-1)

    hist_rows = _make_sc_gather_h(T_h * 5 * B, (T_h + 5) * B)(src_h, idx_h)

    static_out = jnp.concatenate(
        [static_cont_emb[None], stat_rows.reshape(4, B, H)], axis=0
    ).transpose(1, 2, 0)
    hist_out = hist_rows.reshape(T_h, 5, B, H).transpose(2, 0, 3, 1)
    fut_out = fut_rows.reshape(T_f, 3, B, H).transpose(2, 0, 3, 1)

    return (static_out, hist_out, fut_out)


# speedup vs baseline: 9.9091x; 1.3293x over previous
import functools

import jax
import jax.numpy as jnp
from jax import lax
from jax.experimental import pallas as pl
from jax.experimental.pallas import tpu as pltpu
from jax.experimental.pallas import tpu_sc as plsc

NC = 2
NS = 16
NW = NC * NS
H = 128


def _matmul_body(x_ref, w_ref, b_ref, o_ref):
    o_ref[...] = (
        jnp.dot(x_ref[...], w_ref[...], preferred_element_type=jnp.float32)
        + b_ref[...]
    )


def _linear(x, w, b, bm):
    m = x.shape[0]
    return pl.pallas_call(
        _matmul_body,
        grid=(m // bm,),
        in_specs=[
            pl.BlockSpec((bm, 16), lambda i: (i, 0)),
            pl.BlockSpec((16, H), lambda i: (0, 0)),
            pl.BlockSpec((1, H), lambda i: (0, 0)),
        ],
        out_specs=pl.BlockSpec((bm, H), lambda i: (i, 0)),
        out_shape=jax.ShapeDtypeStruct((m, H), jnp.float32),
    )(x, w, b.reshape(1, H))


def _matmul_t_body(x_ref, w_ref, b_ref, o_ref):
    o_ref[0] = lax.dot_general(
        x_ref[0], w_ref[...], (((0,), (0,)), ((), ())),
        preferred_element_type=jnp.float32) + b_ref[...]


def _linear_t(x_t, w, b):
    t_dim, _, b_dim = x_t.shape
    return pl.pallas_call(
        _matmul_t_body,
        grid=(t_dim,),
        in_specs=[
            pl.BlockSpec((1, 16, b_dim), lambda i: (i, 0, 0)),
            pl.BlockSpec((16, H), lambda i: (0, 0)),
            pl.BlockSpec((1, H), lambda i: (0, 0)),
        ],
        out_specs=pl.BlockSpec((1, b_dim, H), lambda i: (i, 0, 0)),
        out_shape=jax.ShapeDtypeStruct((t_dim, b_dim, H), jnp.float32),
    )(x_t, w, b.reshape(1, H))


def _matmul_t_concat_body(x_ref, w_ref, b_ref, t_ref, o_ref, *, nmm):
    pid = pl.program_id(0)

    @pl.when(pid < nmm)
    def _():
        o_ref[0] = lax.dot_general(
            x_ref[0], w_ref[...], (((0,), (0,)), ((), ())),
            preferred_element_type=jnp.float32) + b_ref[...]

    @pl.when(pid >= nmm)
    def _():
        o_ref[...] = t_ref[...]


def _linear_t_concat(x_t, w, b, tab3):
    t_dim, _, b_dim = x_t.shape
    p_dim = tab3.shape[0]
    return pl.pallas_call(
        functools.partial(_matmul_t_concat_body, nmm=t_dim),
        grid=(t_dim + p_dim,),
        in_specs=[
            pl.BlockSpec((1, 16, b_dim),
                         lambda i: (jnp.minimum(i, t_dim - 1), 0, 0)),
            pl.BlockSpec((16, H), lambda i: (0, 0)),
            pl.BlockSpec((1, H), lambda i: (0, 0)),
            pl.BlockSpec((1, b_dim, H),
                         lambda i: (jnp.maximum(i - t_dim, 0), 0, 0)),
        ],
        out_specs=pl.BlockSpec((1, b_dim, H), lambda i: (i, 0, 0)),
        out_shape=jax.ShapeDtypeStruct((t_dim + p_dim, b_dim, H),
                                       jnp.float32),
    )(x_t, w, b.reshape(1, H), tab3)


def _phase_runner2(bufs):
    def run_phase(n, csz, out, out_off, fire, wait_do):
        def o_copy(k, bd, make_only):
            mk = pltpu.make_async_copy if make_only else pltpu.async_copy
            return mk(bd[0].at[pl.ds(0, csz)],
                      out.at[pl.ds(out_off(k), csz)], bd[2])

        def process(k, bd):
            wait_do(k, bd)
            o_copy(k, bd, False)
            o_copy(k, bd, True).wait()

        if n >= 1:
            fire(0, bufs[0])
        if n >= 2:
            fire(1, bufs[1])

        def body(i, carry):
            for par in range(2):
                k = 2 * i + par
                bd = bufs[par]
                process(k, bd)

                @pl.when(k + 2 < n)
                def _():
                    fire(k + 2, bd)
            return carry

        lax.fori_loop(0, n // 2, body, 0)
        if n % 2 == 1:
            process(n - 1, bufs[(n - 1) % 2])

    return run_phase


def _simple_idx_src(src, ixv, csz, bufs):
    def fire(k, bd):
        pltpu.async_copy(src.at[ixv.at[pl.ds(k * csz, csz)]],
                         bd[0].at[pl.ds(0, csz)], bd[1])

    def wait_do(k, bd):
        pltpu.make_async_copy(src.at[ixv.at[pl.ds(k * csz, csz)]],
                              bd[0].at[pl.ds(0, csz)], bd[1]).wait()
    return fire, wait_do


def _phase_runner(wid, bufs):
    def run_phase(rpt, ixv, out, fire, wait_do):
            base = wid * rpt
            n_full, rem = divmod(rpt, 128)

            def o_copy(off, sz, bd, make_only):
                mk = pltpu.make_async_copy if make_only else pltpu.async_copy
                return mk(bd[0].at[pl.ds(0, sz)],
                          out.at[pl.ds(base + off, sz)], bd[2])

            def process(k, sz, bd):
                off = k * 128
                wait_do(off, sz, bd)
                o_copy(off, sz, bd, False)
                o_copy(off, sz, bd, True).wait()

            if n_full >= 1:
                fire(0, 128, bufs[0])
            if n_full >= 2:
                fire(128, 128, bufs[1])

            def body(i, carry):
                for par in range(2):
                    k = 2 * i + par
                    bd = bufs[par]
                    process(k, 128, bd)

                    @pl.when(k + 2 < n_full)
                    def _():
                        fire((k + 2) * 128, 128, bd)
                return carry

            lax.fori_loop(0, n_full // 2, body, 0)

            if n_full % 2 == 1:
                process(n_full - 1, 128, bufs[(n_full - 1) % 2])
            if rem:
                bd = bufs[n_full % 2]
                fire(n_full * 128, rem, bd)
                process(n_full, rem, bd)

    return run_phase


def _simple_src(src, ixv):
    def fire(off, sz, bd):
        pltpu.async_copy(src.at[ixv.at[pl.ds(off, sz)]],
                         bd[0].at[pl.ds(0, sz)], bd[1])

    def wait_do(off, sz, bd):
        pltpu.make_async_copy(src.at[ixv.at[pl.ds(off, sz)]],
                              bd[0].at[pl.ds(0, sz)], bd[1]).wait()
    return fire, wait_do


def _sc_scratch(idx_sizes, shared_rows):
    scratch = [pltpu.VMEM_SHARED((r, H), jnp.float32) for r in shared_rows]
    scratch += [pltpu.VMEM((n,), jnp.int32) for n in idx_sizes]
    for _ in range(2):
        scratch += [pltpu.VMEM((128, H), jnp.float32),
                    pltpu.SemaphoreType.DMA, pltpu.SemaphoreType.DMA]
    return scratch


def _make_sc_gather_fs(n_f, n_s):
    rpt_f, rpt_s = n_f // NW, n_s // NW
    mesh = plsc.VectorSubcoreMesh(core_axis_name="c", subcore_axis_name="s")
    out_type = [jax.ShapeDtypeStruct((n_f, H), jnp.float32),
                jax.ShapeDtypeStruct((n_s, H), jnp.float32)]

    @functools.partial(
        pl.kernel, mesh=mesh, out_type=out_type,
        scratch_types=_sc_scratch([rpt_f, rpt_s], [3000]),
        compiler_params=pltpu.CompilerParams(needs_layout_passes=False))
    def sc_kernel(tab_f, idx_f, tab_s, idx_s, out_f, out_s,
                  shr_f, ixf, ixs, r0, g0, o0, r1, g1, o1):
        sid = lax.axis_index("s")
        wid = sid * NC + lax.axis_index("c")
        bufs = ((r0, g0, o0), (r1, g1, o1))

        @pl.when(sid == 0)
        def _():
            pltpu.sync_copy(tab_f, shr_f)

        pltpu.sync_copy(idx_f.at[pl.ds(wid * rpt_f, rpt_f)], ixf)
        pltpu.sync_copy(idx_s.at[pl.ds(wid * rpt_s, rpt_s)], ixs)
        plsc.subcore_barrier()

        run_phase = _phase_runner(wid, bufs)
        run_phase(rpt_f, ixf, out_f, *_simple_src(shr_f, ixf))
        run_phase(rpt_s, ixs, out_s, *_simple_src(tab_s, ixs))

    return sc_kernel


def _make_sc_gather_h(t_h, b_dim):
    n_h = t_h * 5 * b_dim
    n_cat = t_h * 4 * b_dim
    cat_cpt = (n_cat // 128) // NW
    cont_cpt = (t_h * b_dim // 64) // NW
    assert cat_cpt * NW * 128 == n_cat
    assert cont_cpt * NW * 64 == t_h * b_dim
    mesh = plsc.VectorSubcoreMesh(core_axis_name="c", subcore_axis_name="s")

    @functools.partial(
        pl.kernel, mesh=mesh,
        out_type=jax.ShapeDtypeStruct((n_h, H), jnp.float32),
        scratch_types=_sc_scratch([cat_cpt * 128], [4000]),
        compiler_params=pltpu.CompilerParams(needs_layout_passes=False))
    def sc_kernel(cont_h, tab_h, idx_cat, out_h,
                  shr_h, ixh, r0, g0, o0, r1, g1, o1):
        sid = lax.axis_index("s")
        wid = sid * NC + lax.axis_index("c")
        bufs = ((r0, g0, o0), (r1, g1, o1))

        @pl.when(sid == 0)
        def _():
            pltpu.sync_copy(tab_h, shr_h)

        pltpu.sync_copy(idx_cat.at[pl.ds(wid * cat_cpt * 128,
                                         cat_cpt * 128)], ixh)
        plsc.subcore_barrier()
        run_phase = _phase_runner2(bufs)

        def cat_out_off(k):
            gj = wid * cat_cpt + k
            p = gj // 8
            t = p // 4
            cc = p % 4
            return (t * 5 + cc + 1) * b_dim + (gj % 8) * 128

        run_phase(cat_cpt, 128, out_h, cat_out_off,
                  *_simple_idx_src(shr_h, ixh, 128, bufs))

        def cont_src_off(k):
            gj = wid * cont_cpt + k
            return (gj // 16) * b_dim + (gj % 16) * 64

        def cont_out_off(k):
            gj = wid * cont_cpt + k
            return (gj // 16) * 5 * b_dim + (gj % 16) * 64

        def cont_fire(k, bd):
            pltpu.async_copy(cont_h.at[pl.ds(cont_src_off(k), 64)],
                             bd[0].at[pl.ds(0, 64)], bd[1])

        def cont_wait(k, bd):
            pltpu.make_async_copy(cont_h.at[pl.ds(cont_src_off(k), 64)],
                                  bd[0].at[pl.ds(0, 64)], bd[1]).wait()

        run_phase(cont_cpt, 64, out_h, cont_out_off, cont_fire, cont_wait)

    return sc_kernel


def kernel(static_cont_input, static_cat_input, history_cont_input,
           history_cat_input, future_input, W_s, b_s, W_h, b_h,
           static_tables, history_tables, future_tables):
    B, T_h, _ = history_cont_input.shape
    T_f = future_input.shape[1]
    i32 = jnp.int32

    tab_s = static_tables.reshape(4 * 10000, H)
    tab_h = history_tables.reshape(4 * 1000, H)
    tab_f = future_tables.reshape(3 * 1000, H)

    idx_f = (future_input.astype(i32).transpose(1, 2, 0)
             + (jnp.arange(3, dtype=i32) * 1000).reshape(1, 3, 1)
             ).reshape(-1)
    idx_s = (static_cat_input.astype(i32).T
             + (jnp.arange(4, dtype=i32) * 10000).reshape(4, 1)
             ).reshape(-1)

    fut_rows, stat_rows = _make_sc_gather_fs(T_f * 3 * B, 4 * B)(
        tab_f, idx_f, tab_s, idx_s)

    static_cont_emb = _linear(static_cont_input, W_s, b_s, bm=B)
    cont_h = _linear_t(history_cont_input.transpose(1, 2, 0),
                       W_h, b_h).reshape(T_h * B, H)

    idx_cat = (history_cat_input.astype(i32).transpose(1, 2, 0)
               + (jnp.arange(4, dtype=i32) * 1000).reshape(1, 4, 1)
               ).reshape(-1)

    hist_rows = _make_sc_gather_h(T_h, B)(cont_h, tab_h, idx_cat)

    static_out = jnp.concatenate(
        [static_cont_emb[None], stat_rows.reshape(4, B, H)], axis=0
    ).transpose(1, 2, 0)
    hist_out = hist_rows.reshape(T_h, 5, B, H).transpose(2, 0, 3, 1)
    fut_out = fut_rows.reshape(T_f, 3, B, H).transpose(2, 0, 3, 1)

    return (static_out, hist_out, fut_out)


# speedup vs baseline: 11.1292x; 1.1231x over previous
import functools

import jax
import jax.numpy as jnp
from jax import lax
from jax.experimental import pallas as pl
from jax.experimental.pallas import tpu as pltpu
from jax.experimental.pallas import tpu_sc as plsc

NC = 2
NS = 16
NW = NC * NS
H = 128


def _matmul_body(x_ref, w_ref, b_ref, o_ref):
    o_ref[...] = (
        jnp.dot(x_ref[...], w_ref[...], preferred_element_type=jnp.float32)
        + b_ref[...]
    )


def _linear(x, w, b, bm):
    m = x.shape[0]
    return pl.pallas_call(
        _matmul_body,
        grid=(m // bm,),
        in_specs=[
            pl.BlockSpec((bm, 16), lambda i: (i, 0)),
            pl.BlockSpec((16, H), lambda i: (0, 0)),
            pl.BlockSpec((1, H), lambda i: (0, 0)),
        ],
        out_specs=pl.BlockSpec((bm, H), lambda i: (i, 0)),
        out_shape=jax.ShapeDtypeStruct((m, H), jnp.float32),
    )(x, w, b.reshape(1, H))


def _matmul_t_body(x_ref, w_ref, b_ref, o_ref, *, bt):
    for j in range(bt):
        o_ref[j] = lax.dot_general(
            x_ref[j], w_ref[...], (((0,), (0,)), ((), ())),
            preferred_element_type=jnp.float32) + b_ref[...]


def _linear_t(x_t, w, b, bt):
    t_dim, _, b_dim = x_t.shape
    return pl.pallas_call(
        functools.partial(_matmul_t_body, bt=bt),
        grid=(t_dim // bt,),
        in_specs=[
            pl.BlockSpec((bt, 16, b_dim), lambda i: (i, 0, 0)),
            pl.BlockSpec((16, H), lambda i: (0, 0)),
            pl.BlockSpec((1, H), lambda i: (0, 0)),
        ],
        out_specs=pl.BlockSpec((bt, b_dim, H), lambda i: (i, 0, 0)),
        out_shape=jax.ShapeDtypeStruct((t_dim, b_dim, H), jnp.float32),
    )(x_t, w, b.reshape(1, H))


def _matmul_t_concat_body(x_ref, w_ref, b_ref, t_ref, o_ref, *, nmm):
    pid = pl.program_id(0)

    @pl.when(pid < nmm)
    def _():
        o_ref[0] = lax.dot_general(
            x_ref[0], w_ref[...], (((0,), (0,)), ((), ())),
            preferred_element_type=jnp.float32) + b_ref[...]

    @pl.when(pid >= nmm)
    def _():
        o_ref[...] = t_ref[...]


def _linear_t_concat(x_t, w, b, tab3):
    t_dim, _, b_dim = x_t.shape
    p_dim = tab3.shape[0]
    return pl.pallas_call(
        functools.partial(_matmul_t_concat_body, nmm=t_dim),
        grid=(t_dim + p_dim,),
        in_specs=[
            pl.BlockSpec((1, 16, b_dim),
                         lambda i: (jnp.minimum(i, t_dim - 1), 0, 0)),
            pl.BlockSpec((16, H), lambda i: (0, 0)),
            pl.BlockSpec((1, H), lambda i: (0, 0)),
            pl.BlockSpec((1, b_dim, H),
                         lambda i: (jnp.maximum(i - t_dim, 0), 0, 0)),
        ],
        out_specs=pl.BlockSpec((1, b_dim, H), lambda i: (i, 0, 0)),
        out_shape=jax.ShapeDtypeStruct((t_dim + p_dim, b_dim, H),
                                       jnp.float32),
    )(x_t, w, b.reshape(1, H), tab3)


def _phase_runner2(bufs):
    def run_phase(n, csz, out, out_off, fire, wait_do):
        def o_copy(k, bd, make_only):
            mk = pltpu.make_async_copy if make_only else pltpu.async_copy
            return mk(bd[0].at[pl.ds(0, csz)],
                      out.at[pl.ds(out_off(k), csz)], bd[2])

        def process(k, bd):
            wait_do(k, bd)
            o_copy(k, bd, False)
            o_copy(k, bd, True).wait()

        if n >= 1:
            fire(0, bufs[0])
        if n >= 2:
            fire(1, bufs[1])

        def body(i, carry):
            for par in range(2):
                k = 2 * i + par
                bd = bufs[par]
                process(k, bd)

                @pl.when(k + 2 < n)
                def _():
                    fire(k + 2, bd)
            return carry

        lax.fori_loop(0, n // 2, body, 0)
        if n % 2 == 1:
            process(n - 1, bufs[(n - 1) % 2])

    return run_phase


def _simple_idx_src(src, ixv, csz, bufs):
    def fire(k, bd):
        pltpu.async_copy(src.at[ixv.at[pl.ds(k * csz, csz)]],
                         bd[0].at[pl.ds(0, csz)], bd[1])

    def wait_do(k, bd):
        pltpu.make_async_copy(src.at[ixv.at[pl.ds(k * csz, csz)]],
                              bd[0].at[pl.ds(0, csz)], bd[1]).wait()
    return fire, wait_do


def _phase_runner(wid, bufs):
    def run_phase(rpt, ixv, out, fire, wait_do):
            base = wid * rpt
            n_full, rem = divmod(rpt, 128)

            def o_copy(off, sz, bd, make_only):
                mk = pltpu.make_async_copy if make_only else pltpu.async_copy
                return mk(bd[0].at[pl.ds(0, sz)],
                          out.at[pl.ds(base + off, sz)], bd[2])

            def process(k, sz, bd):
                off = k * 128
                wait_do(off, sz, bd)
                o_copy(off, sz, bd, False)
                o_copy(off, sz, bd, True).wait()

            if n_full >= 1:
                fire(0, 128, bufs[0])
            if n_full >= 2:
                fire(128, 128, bufs[1])

            def body(i, carry):
                for par in range(2):
                    k = 2 * i + par
                    bd = bufs[par]
                    process(k, 128, bd)

                    @pl.when(k + 2 < n_full)
                    def _():
                        fire((k + 2) * 128, 128, bd)
                return carry

            lax.fori_loop(0, n_full // 2, body, 0)

            if n_full % 2 == 1:
                process(n_full - 1, 128, bufs[(n_full - 1) % 2])
            if rem:
                bd = bufs[n_full % 2]
                fire(n_full * 128, rem, bd)
                process(n_full, rem, bd)

    return run_phase


def _simple_src(src, ixv):
    def fire(off, sz, bd):
        pltpu.async_copy(src.at[ixv.at[pl.ds(off, sz)]],
                         bd[0].at[pl.ds(0, sz)], bd[1])

    def wait_do(off, sz, bd):
        pltpu.make_async_copy(src.at[ixv.at[pl.ds(off, sz)]],
                              bd[0].at[pl.ds(0, sz)], bd[1]).wait()
    return fire, wait_do


def _sc_scratch(idx_sizes, shared_rows):
    scratch = [pltpu.VMEM_SHARED((r, H), jnp.float32) for r in shared_rows]
    scratch += [pltpu.VMEM((n,), jnp.int32) for n in idx_sizes]
    for _ in range(2):
        scratch += [pltpu.VMEM((128, H), jnp.float32),
                    pltpu.SemaphoreType.DMA, pltpu.SemaphoreType.DMA]
    return scratch


def _make_sc_gather_fs(n_f, n_s):
    rpt_f, rpt_s = n_f // NW, n_s // NW
    mesh = plsc.VectorSubcoreMesh(core_axis_name="c", subcore_axis_name="s")
    out_type = [jax.ShapeDtypeStruct((n_f, H), jnp.float32),
                jax.ShapeDtypeStruct((n_s, H), jnp.float32)]

    @functools.partial(
        pl.kernel, mesh=mesh, out_type=out_type,
        scratch_types=_sc_scratch([rpt_f, rpt_s], [3000]),
        compiler_params=pltpu.CompilerParams(needs_layout_passes=False))
    def sc_kernel(tab_f, idx_f, tab_s, idx_s, out_f, out_s,
                  shr_f, ixf, ixs, r0, g0, o0, r1, g1, o1):
        sid = lax.axis_index("s")
        wid = sid * NC + lax.axis_index("c")
        bufs = ((r0, g0, o0), (r1, g1, o1))

        @pl.when(sid == 0)
        def _():
            pltpu.sync_copy(tab_f, shr_f)

        pltpu.sync_copy(idx_f.at[pl.ds(wid * rpt_f, rpt_f)], ixf)
        pltpu.sync_copy(idx_s.at[pl.ds(wid * rpt_s, rpt_s)], ixs)
        plsc.subcore_barrier()

        run_phase = _phase_runner(wid, bufs)
        run_phase(rpt_f, ixf, out_f, *_simple_src(shr_f, ixf))
        run_phase(rpt_s, ixs, out_s, *_simple_src(tab_s, ixs))

    return sc_kernel


def _make_sc_gather_h(t_h, b_dim):
    n_h = t_h * 5 * b_dim
    n_cat = t_h * 4 * b_dim
    cat_cpt = (n_cat // 128) // NW
    cont_cpt = (t_h * b_dim // 64) // NW
    assert cat_cpt * NW * 128 == n_cat
    assert cont_cpt * NW * 64 == t_h * b_dim
    mesh = plsc.VectorSubcoreMesh(core_axis_name="c", subcore_axis_name="s")

    @functools.partial(
        pl.kernel, mesh=mesh,
        out_type=jax.ShapeDtypeStruct((n_h, H), jnp.float32),
        scratch_types=_sc_scratch([cat_cpt * 128], [4000]),
        compiler_params=pltpu.CompilerParams(needs_layout_passes=False))
    def sc_kernel(cont_h, tab_h, idx_cat, out_h,
                  shr_h, ixh, r0, g0, o0, r1, g1, o1):
        sid = lax.axis_index("s")
        wid = sid * NC + lax.axis_index("c")
        bufs = ((r0, g0, o0), (r1, g1, o1))

        @pl.when(sid == 0)
        def _():
            pltpu.sync_copy(tab_h, shr_h)

        pltpu.sync_copy(idx_cat.at[pl.ds(wid * cat_cpt * 128,
                                         cat_cpt * 128)], ixh)
        plsc.subcore_barrier()
        run_phase = _phase_runner2(bufs)

        def cat_out_off(k):
            gj = wid * cat_cpt + k
            p = gj // 8
            t = p // 4
            cc = p % 4
            return (t * 5 + cc + 1) * b_dim + (gj % 8) * 128

        run_phase(cat_cpt, 128, out_h, cat_out_off,
                  *_simple_idx_src(shr_h, ixh, 128, bufs))

        def cont_src_off(k):
            gj = wid * cont_cpt + k
            return (gj // 16) * b_dim + (gj % 16) * 64

        def cont_out_off(k):
            gj = wid * cont_cpt + k
            return (gj // 16) * 5 * b_dim + (gj % 16) * 64

        def cont_fire(k, bd):
            pltpu.async_copy(cont_h.at[pl.ds(cont_src_off(k), 64)],
                             bd[0].at[pl.ds(0, 64)], bd[1])

        def cont_wait(k, bd):
            pltpu.make_async_copy(cont_h.at[pl.ds(cont_src_off(k), 64)],
                                  bd[0].at[pl.ds(0, 64)], bd[1]).wait()

        run_phase(cont_cpt, 64, out_h, cont_out_off, cont_fire, cont_wait)

    return sc_kernel


def kernel(static_cont_input, static_cat_input, history_cont_input,
           history_cat_input, future_input, W_s, b_s, W_h, b_h,
           static_tables, history_tables, future_tables):
    B, T_h, _ = history_cont_input.shape
    T_f = future_input.shape[1]
    i32 = jnp.int32

    tab_s = static_tables.reshape(4 * 10000, H)
    tab_h = history_tables.reshape(4 * 1000, H)
    tab_f = future_tables.reshape(3 * 1000, H)

    idx_f = (future_input.astype(i32).transpose(1, 2, 0)
             + (jnp.arange(3, dtype=i32) * 1000).reshape(1, 3, 1)
             ).reshape(-1)
    idx_s = (static_cat_input.astype(i32).T
             + (jnp.arange(4, dtype=i32) * 10000).reshape(4, 1)
             ).reshape(-1)

    fut_rows, stat_rows = _make_sc_gather_fs(T_f * 3 * B, 4 * B)(
        tab_f, idx_f, tab_s, idx_s)

    static_cont_emb = _linear(static_cont_input, W_s, b_s, bm=B)
    cont_h = _linear_t(history_cont_input.transpose(1, 2, 0),
                       W_h, b_h, bt=5).reshape(T_h * B, H)

    idx_cat = (history_cat_input.astype(i32).transpose(1, 2, 0)
               + (jnp.arange(4, dtype=i32) * 1000).reshape(1, 4, 1)
               ).reshape(-1)

    hist_rows = _make_sc_gather_h(T_h, B)(cont_h, tab_h, idx_cat)

    static_out = jnp.concatenate(
        [static_cont_emb[None], stat_rows.reshape(4, B, H)], axis=0
    ).transpose(1, 2, 0)
    hist_out = hist_rows.reshape(T_h, 5, B, H).transpose(2, 0, 3, 1)
    fut_out = fut_rows.reshape(T_f, 3, B, H).transpose(2, 0, 3, 1)

    return (static_out, hist_out, fut_out)


# speedup vs baseline: 11.3589x; 1.0206x over previous
import functools

import jax
import jax.numpy as jnp
from jax import lax
from jax.experimental import pallas as pl
from jax.experimental.pallas import tpu as pltpu
from jax.experimental.pallas import tpu_sc as plsc

NC = 2
NS = 16
NW = NC * NS
H = 128


def _matmul_body(x_ref, w_ref, b_ref, o_ref):
    o_ref[...] = (
        jnp.dot(x_ref[...], w_ref[...], preferred_element_type=jnp.float32)
        + b_ref[...]
    )


def _linear(x, w, b, bm):
    m = x.shape[0]
    return pl.pallas_call(
        _matmul_body,
        grid=(m // bm,),
        in_specs=[
            pl.BlockSpec((bm, 16), lambda i: (i, 0)),
            pl.BlockSpec((16, H), lambda i: (0, 0)),
            pl.BlockSpec((1, H), lambda i: (0, 0)),
        ],
        out_specs=pl.BlockSpec((bm, H), lambda i: (i, 0)),
        out_shape=jax.ShapeDtypeStruct((m, H), jnp.float32),
    )(x, w, b.reshape(1, H))


def _matmul_t_body(x_ref, w_ref, b_ref, o_ref, *, bt):
    for j in range(bt):
        o_ref[j] = lax.dot_general(
            x_ref[j], w_ref[...], (((0,), (0,)), ((), ())),
            preferred_element_type=jnp.float32) + b_ref[...]


def _linear_t(x_t, w, b, bt):
    t_dim, _, b_dim = x_t.shape
    return pl.pallas_call(
        functools.partial(_matmul_t_body, bt=bt),
        grid=(t_dim // bt,),
        in_specs=[
            pl.BlockSpec((bt, 16, b_dim), lambda i: (i, 0, 0)),
            pl.BlockSpec((16, H), lambda i: (0, 0)),
            pl.BlockSpec((1, H), lambda i: (0, 0)),
        ],
        out_specs=pl.BlockSpec((bt, b_dim, H), lambda i: (i, 0, 0)),
        out_shape=jax.ShapeDtypeStruct((t_dim, b_dim, H), jnp.float32),
    )(x_t, w, b.reshape(1, H))


def _matmul_t_concat_body(x_ref, w_ref, b_ref, t_ref, o_ref, *, nmm):
    pid = pl.program_id(0)

    @pl.when(pid < nmm)
    def _():
        o_ref[0] = lax.dot_general(
            x_ref[0], w_ref[...], (((0,), (0,)), ((), ())),
            preferred_element_type=jnp.float32) + b_ref[...]

    @pl.when(pid >= nmm)
    def _():
        o_ref[...] = t_ref[...]


def _linear_t_concat(x_t, w, b, tab3):
    t_dim, _, b_dim = x_t.shape
    p_dim = tab3.shape[0]
    return pl.pallas_call(
        functools.partial(_matmul_t_concat_body, nmm=t_dim),
        grid=(t_dim + p_dim,),
        in_specs=[
            pl.BlockSpec((1, 16, b_dim),
                         lambda i: (jnp.minimum(i, t_dim - 1), 0, 0)),
            pl.BlockSpec((16, H), lambda i: (0, 0)),
            pl.BlockSpec((1, H), lambda i: (0, 0)),
            pl.BlockSpec((1, b_dim, H),
                         lambda i: (jnp.maximum(i - t_dim, 0), 0, 0)),
        ],
        out_specs=pl.BlockSpec((1, b_dim, H), lambda i: (i, 0, 0)),
        out_shape=jax.ShapeDtypeStruct((t_dim + p_dim, b_dim, H),
                                       jnp.float32),
    )(x_t, w, b.reshape(1, H), tab3)


def _phase_runner2(bufs):
    def run_phase(n, csz, out, out_off, fire, wait_do):
        def o_copy(k, bd, make_only):
            mk = pltpu.make_async_copy if make_only else pltpu.async_copy
            return mk(bd[0].at[pl.ds(0, csz)],
                      out.at[pl.ds(out_off(k), csz)], bd[2])

        def step(k, j):
            wait_do(k, bufs[j])
            o_copy(k, bufs[j], False)

        if n >= 1:
            fire(0, bufs[0])
        if n >= 2:
            fire(1, bufs[1])

        n_main = (n // 4) * 4

        def body(i, carry):
            for j in range(4):
                k = 4 * i + j
                step(k, j)

                @pl.when(k + 2 < n)
                def _():
                    bd2 = bufs[(j + 2) % 4]

                    @pl.when(k >= 2)
                    def _():
                        o_copy(jnp.maximum(k - 2, 0), bd2, True).wait()

                    fire(k + 2, bd2)
            return carry

        lax.fori_loop(0, n // 4, body, 0)
        for k in range(n_main, n):
            j = k % 4
            step(k, j)
            if k + 2 < n:
                bd2 = bufs[(j + 2) % 4]
                if k >= 2:
                    o_copy(k - 2, bd2, True).wait()
                fire(k + 2, bd2)
        for k in range(max(n - 4, 0), n):
            o_copy(k, bufs[k % 4], True).wait()

    return run_phase


def _simple_idx_src(src, ixv, csz, bufs):
    def fire(k, bd):
        pltpu.async_copy(src.at[ixv.at[pl.ds(k * csz, csz)]],
                         bd[0].at[pl.ds(0, csz)], bd[1])

    def wait_do(k, bd):
        pltpu.make_async_copy(src.at[ixv.at[pl.ds(k * csz, csz)]],
                              bd[0].at[pl.ds(0, csz)], bd[1]).wait()
    return fire, wait_do


def _phase_runner(wid, bufs):
    def run_phase(rpt, ixv, out, fire, wait_do):
            base = wid * rpt
            n_full, rem = divmod(rpt, 128)

            def o_copy(off, sz, bd, make_only):
                mk = pltpu.make_async_copy if make_only else pltpu.async_copy
                return mk(bd[0].at[pl.ds(0, sz)],
                          out.at[pl.ds(base + off, sz)], bd[2])

            def process(k, sz, bd):
                off = k * 128
                wait_do(off, sz, bd)
                o_copy(off, sz, bd, False)
                o_copy(off, sz, bd, True).wait()

            if n_full >= 1:
                fire(0, 128, bufs[0])
            if n_full >= 2:
                fire(128, 128, bufs[1])

            def body(i, carry):
                for par in range(2):
                    k = 2 * i + par
                    bd = bufs[par]
                    process(k, 128, bd)

                    @pl.when(k + 2 < n_full)
                    def _():
                        fire((k + 2) * 128, 128, bd)
                return carry

            lax.fori_loop(0, n_full // 2, body, 0)

            if n_full % 2 == 1:
                process(n_full - 1, 128, bufs[(n_full - 1) % 2])
            if rem:
                bd = bufs[n_full % 2]
                fire(n_full * 128, rem, bd)
                process(n_full, rem, bd)

    return run_phase


def _simple_src(src, ixv):
    def fire(off, sz, bd):
        pltpu.async_copy(src.at[ixv.at[pl.ds(off, sz)]],
                         bd[0].at[pl.ds(0, sz)], bd[1])

    def wait_do(off, sz, bd):
        pltpu.make_async_copy(src.at[ixv.at[pl.ds(off, sz)]],
                              bd[0].at[pl.ds(0, sz)], bd[1]).wait()
    return fire, wait_do


NBUF = 4


def _sc_scratch(idx_sizes, shared_rows):
    scratch = [pltpu.VMEM_SHARED((r, H), jnp.float32) for r in shared_rows]
    scratch += [pltpu.VMEM((n,), jnp.int32) for n in idx_sizes]
    for _ in range(NBUF):
        scratch += [pltpu.VMEM((128, H), jnp.float32),
                    pltpu.SemaphoreType.DMA, pltpu.SemaphoreType.DMA]
    return scratch


def _make_sc_gather_fs(n_f, n_s):
    rpt_f, rpt_s = n_f // NW, n_s // NW
    mesh = plsc.VectorSubcoreMesh(core_axis_name="c", subcore_axis_name="s")
    out_type = [jax.ShapeDtypeStruct((n_f, H), jnp.float32),
                jax.ShapeDtypeStruct((n_s, H), jnp.float32)]

    @functools.partial(
        pl.kernel, mesh=mesh, out_type=out_type,
        scratch_types=_sc_scratch([rpt_f, rpt_s], [3000]),
        compiler_params=pltpu.CompilerParams(needs_layout_passes=False))
    def sc_kernel(tab_f, idx_f, tab_s, idx_s, out_f, out_s,
                  shr_f, ixf, ixs, *bs):
        sid = lax.axis_index("s")
        wid = sid * NC + lax.axis_index("c")
        bufs = tuple(bs[3 * i: 3 * i + 3] for i in range(NBUF))

        @pl.when(sid == 0)
        def _():
            pltpu.sync_copy(tab_f, shr_f)

        pltpu.sync_copy(idx_f.at[pl.ds(wid * rpt_f, rpt_f)], ixf)
        pltpu.sync_copy(idx_s.at[pl.ds(wid * rpt_s, rpt_s)], ixs)
        plsc.subcore_barrier()

        run_phase = _phase_runner2(bufs)
        run_phase(rpt_f // 128, 128, out_f,
                  lambda k: wid * rpt_f + k * 128,
                  *_simple_idx_src(shr_f, ixf, 128, bufs))
        run_phase(rpt_s // 128, 128, out_s,
                  lambda k: wid * rpt_s + k * 128,
                  *_simple_idx_src(tab_s, ixs, 128, bufs))

    return sc_kernel


def _make_sc_gather_h(t_h, b_dim):
    n_h = t_h * 5 * b_dim
    n_cat = t_h * 4 * b_dim
    cat_cpt = (n_cat // 128) // NW
    cont_cpt = (t_h * b_dim // 64) // NW
    assert cat_cpt * NW * 128 == n_cat
    assert cont_cpt * NW * 64 == t_h * b_dim
    mesh = plsc.VectorSubcoreMesh(core_axis_name="c", subcore_axis_name="s")

    @functools.partial(
        pl.kernel, mesh=mesh,
        out_type=jax.ShapeDtypeStruct((n_h, H), jnp.float32),
        scratch_types=_sc_scratch([cat_cpt * 128], [4000]),
        compiler_params=pltpu.CompilerParams(needs_layout_passes=False))
    def sc_kernel(cont_h, tab_h, idx_cat, out_h, shr_h, ixh, *bs):
        sid = lax.axis_index("s")
        wid = sid * NC + lax.axis_index("c")
        bufs = tuple(bs[3 * i: 3 * i + 3] for i in range(NBUF))

        @pl.when(sid == 0)
        def _():
            pltpu.sync_copy(tab_h, shr_h)

        pltpu.sync_copy(idx_cat.at[pl.ds(wid * cat_cpt * 128,
                                         cat_cpt * 128)], ixh)
        plsc.subcore_barrier()
        run_phase = _phase_runner2(bufs)

        def cat_out_off(k):
            gj = wid * cat_cpt + k
            p = gj // 8
            t = p // 4
            cc = p % 4
            return (t * 5 + cc + 1) * b_dim + (gj % 8) * 128

        run_phase(cat_cpt, 128, out_h, cat_out_off,
                  *_simple_idx_src(shr_h, ixh, 128, bufs))

        def cont_src_off(k):
            gj = wid * cont_cpt + k
            return (gj // 16) * b_dim + (gj % 16) * 64

        def cont_out_off(k):
            gj = wid * cont_cpt + k
            return (gj // 16) * 5 * b_dim + (gj % 16) * 64

        def cont_fire(k, bd):
            pltpu.async_copy(cont_h.at[pl.ds(cont_src_off(k), 64)],
                             bd[0].at[pl.ds(0, 64)], bd[1])

        def cont_wait(k, bd):
            pltpu.make_async_copy(cont_h.at[pl.ds(cont_src_off(k), 64)],
                                  bd[0].at[pl.ds(0, 64)], bd[1]).wait()

        run_phase(cont_cpt, 64, out_h, cont_out_off, cont_fire, cont_wait)

    return sc_kernel


def kernel(static_cont_input, static_cat_input, history_cont_input,
           history_cat_input, future_input, W_s, b_s, W_h, b_h,
           static_tables, history_tables, future_tables):
    B, T_h, _ = history_cont_input.shape
    T_f = future_input.shape[1]
    i32 = jnp.int32

    tab_s = static_tables.reshape(4 * 10000, H)
    tab_h = history_tables.reshape(4 * 1000, H)
    tab_f = future_tables.reshape(3 * 1000, H)

    idx_f = (future_input.astype(i32).transpose(1, 2, 0)
             + (jnp.arange(3, dtype=i32) * 1000).reshape(1, 3, 1)
             ).reshape(-1)
    idx_s = (static_cat_input.astype(i32).T
             + (jnp.arange(4, dtype=i32) * 10000).reshape(4, 1)
             ).reshape(-1)

    fut_rows, stat_rows = _make_sc_gather_fs(T_f * 3 * B, 4 * B)(
        tab_f, idx_f, tab_s, idx_s)

    static_cont_emb = _linear(static_cont_input, W_s, b_s, bm=B)
    cont_h = _linear_t(history_cont_input.transpose(1, 2, 0),
                       W_h, b_h, bt=5).reshape(T_h * B, H)

    idx_cat = (history_cat_input.astype(i32).transpose(1, 2, 0)
               + (jnp.arange(4, dtype=i32) * 1000).reshape(1, 4, 1)
               ).reshape(-1)

    hist_rows = _make_sc_gather_h(T_h, B)(cont_h, tab_h, idx_cat)

    static_out = jnp.concatenate(
        [static_cont_emb[None], stat_rows.reshape(4, B, H)], axis=0
    ).transpose(1, 2, 0)
    hist_out = hist_rows.reshape(T_h, 5, B, H).transpose(2, 0, 3, 1)
    fut_out = fut_rows.reshape(T_f, 3, B, H).transpose(2, 0, 3, 1)

    return (static_out, hist_out, fut_out)
